# baseline (speedup 1.0000x reference)
import functools

import jax
import jax.numpy as jnp
import numpy as np
from jax import lax
from jax.experimental import pallas as pl
from jax.experimental.pallas import tpu as pltpu
from jax.experimental.pallas import tpu_sc as plsc

_B, _N, _C = 32, 8192, 64
_NPOINTS = 2048
_SROWS = 4
_U0, _U1 = 11, 5
_NT0 = 2 * _U0
_NT1 = 2 * _U1

_IDX_CONST = None


def _index_consts() -> np.ndarray:
    global _IDX_CONST
    if _IDX_CONST is None:
        with jax.ensure_compile_time_eval():
            idx = jax.random.randint(
                jax.random.key(42), (_B, _NPOINTS), 0, _N, dtype=jnp.int32)
            _IDX_CONST = np.asarray(idx)
    return _IDX_CONST


def _sampler_body(table_hbm, idx_hbm, out_hbm, idx_v, inbuf, outbuf,
                  isem0, isem1, isem2, osem0, osem1, xsem0, xsem1):
    isems = (isem0, isem1, isem2)
    osems = (osem0, osem1)
    xsems = (xsem0, xsem1)
    core = lax.axis_index("c")
    sub = lax.axis_index("s")
    on0 = core == 0
    on1 = core != 0
    base_u = jnp.where(on0, sub * _U0, 176 + sub * _U1)
    rsplats = [jnp.full((16,), r, jnp.int32) for r in range(_SROWS)]

    def make_in(t):
        row0 = 8 * base_u + 4 * t
        return pltpu.make_async_copy(
            table_hbm.at[pl.ds(row0, _SROWS)], inbuf.at[t % 3], isems[t % 3])

    def make_idx(u):
        return pltpu.make_async_copy(
            idx_hbm.at[pl.ds(((base_u + u) >> 3) * _NPOINTS, _NPOINTS)],
            idx_v.at[pl.ds((u % 2) * _NPOINTS, _NPOINTS)], xsems[u % 2])

    def make_out(t):
        row0 = 8 * base_u + 4 * t
        return pltpu.make_async_copy(
            outbuf.at[t % 2], out_hbm.at[pl.ds(row0, _SROWS)], osems[t % 2])

    ih = {t: make_in(t) for t in range(_NT0)}
    oh = {t: make_out(t) for t in range(_NT0)}
    xh = {u: make_idx(u) for u in range(_U0)}

    def gather(t):
        src = inbuf.at[t % 3]
        dst = outbuf.at[t % 2]
        xoff = ((t // 2) % 2) * _NPOINTS

        def body(i, carry):
            base = xoff + i * 128
            idxvs = [idx_v[pl.ds(base + u * 16, 16)] for u in range(8)]
            vals = [plsc.load_gather(src, [rsplats[r], idxvs[u]])
                    for u in range(8) for r in range(_SROWS)]
            obase = i * 128
            for u in range(8):
                for r in range(_SROWS):
                    dst[r, pl.ds(obase + u * 16, 16)] = vals[u * _SROWS + r]
            return carry

        lax.fori_loop(0, _NPOINTS // 128, body, 0)

    xh[0].start()
    for t in range(3):
        ih[t].start()

    def step(t):
        ih[t].wait()
        if t % 2 == 0:
            u = t // 2
            xh[u].wait()
            if u + 1 < _U0:
                if 2 * (u + 1) < _NT1:
                    xh[u + 1].start()
                else:
                    @pl.when(on0)
                    def _():
                        xh[u + 1].start()
        if t >= 2:
            oh[t - 2].wait()
        gather(t)
        if t + 3 < _NT0:
            if t + 3 < _NT1:
                ih[t + 3].start()
            else:
                @pl.when(on0)
                def _():
                    ih[t + 3].start()
        oh[t].start()

    for t in range(_NT1):
        step(t)

    @pl.when(on1)
    def _():
        oh[_NT1 - 2].wait()
        oh[_NT1 - 1].wait()

    @pl.when(on0)
    def _():
        for t in range(_NT1, _NT0):
            step(t)
        oh[_NT0 - 2].wait()
        oh[_NT0 - 1].wait()


@functools.partial(jax.jit, static_argnames=())
def _sampler(table, idx):
    mesh = plsc.VectorSubcoreMesh(core_axis_name="c", subcore_axis_name="s")
    call = pl.kernel(
        _sampler_body,
        out_type=jax.ShapeDtypeStruct((_B * _C, _NPOINTS), jnp.float32),
        mesh=mesh,
        scratch_types=[
            pltpu.VMEM((2 * _NPOINTS,), jnp.int32),
            pltpu.VMEM((3, _SROWS, _N), jnp.float32),
            pltpu.VMEM((2, _SROWS, _NPOINTS), jnp.float32),
            pltpu.SemaphoreType.DMA,
            pltpu.SemaphoreType.DMA,
            pltpu.SemaphoreType.DMA,
            pltpu.SemaphoreType.DMA,
            pltpu.SemaphoreType.DMA,
            pltpu.SemaphoreType.DMA,
            pltpu.SemaphoreType.DMA,
        ],
        compiler_params=pltpu.CompilerParams(needs_layout_passes=False),
    )
    return call(table, idx)


def kernel(inputs):
    table = jnp.transpose(inputs, (0, 2, 1)).reshape(_B * _C, _N)
    idx = jnp.asarray(_index_consts()).reshape(_B * _NPOINTS)
    out = _sampler(table, idx)
    return jnp.transpose(out.reshape(_B, _C, _NPOINTS), (0, 2, 1))


# speedup vs baseline: 1.0132x; 1.0132x over previous
import functools

import jax
import jax.numpy as jnp
import numpy as np
from jax import lax
from jax.experimental import pallas as pl
from jax.experimental.pallas import tpu as pltpu
from jax.experimental.pallas import tpu_sc as plsc

_B, _N, _C = 32, 8192, 64
_NPOINTS = 2048
_SROWS = 4
_U0, _U1 = 11, 5
_NT0 = 2 * _U0
_NT1 = 2 * _U1

_IDX_CONST = None


def _index_consts() -> np.ndarray:
    global _IDX_CONST
    if _IDX_CONST is None:
        with jax.ensure_compile_time_eval():
            idx = jax.random.randint(
                jax.random.key(42), (_B, _NPOINTS), 0, _N, dtype=jnp.int32)
            _IDX_CONST = np.asarray(idx)
    return _IDX_CONST


def _sampler_body(table_hbm, idx_hbm, out_hbm, idx_v, inbuf, outbuf,
                  isem0, isem1, isem2, osem0, osem1, xsem0, xsem1):
    isems = (isem0, isem1, isem2)
    osems = (osem0, osem1)
    xsems = (xsem0, xsem1)
    core = lax.axis_index("c")
    sub = lax.axis_index("s")
    heavy = core != 0
    light = core == 0
    base_u = jnp.where(heavy, sub * _U0, 176 + sub * _U1)
    rsplats = [jnp.full((16,), r, jnp.int32) for r in range(_SROWS)]

    def make_in(t):
        row0 = 8 * base_u + 4 * t
        return pltpu.make_async_copy(
            table_hbm.at[pl.ds(row0, _SROWS)], inbuf.at[t % 3], isems[t % 3])

    def make_idx(u):
        return pltpu.make_async_copy(
            idx_hbm.at[pl.ds(((base_u + u) >> 3) * _NPOINTS, _NPOINTS)],
            idx_v.at[pl.ds((u % 2) * _NPOINTS, _NPOINTS)], xsems[u % 2])

    def make_out(t):
        row0 = 8 * base_u + 4 * t
        return pltpu.make_async_copy(
            outbuf.at[t % 2], out_hbm.at[pl.ds(row0, _SROWS)], osems[t % 2])

    ih = {t: make_in(t) for t in range(_NT0)}
    oh = {t: make_out(t) for t in range(_NT0)}
    xh = {u: make_idx(u) for u in range(_U0)}

    def gather(t):
        src = inbuf.at[t % 3]
        dst = outbuf.at[t % 2]
        xoff = ((t // 2) % 2) * _NPOINTS

        def body(i, carry):
            base = xoff + i * 128
            idxvs = [idx_v[pl.ds(base + u * 16, 16)] for u in range(8)]
            vals = [plsc.load_gather(src, [rsplats[r], idxvs[u]])
                    for u in range(8) for r in range(_SROWS)]
            obase = i * 128
            for u in range(8):
                for r in range(_SROWS):
                    dst[r, pl.ds(obase + u * 16, 16)] = vals[u * _SROWS + r]
            return carry

        lax.fori_loop(0, _NPOINTS // 128, body, 0)

    xh[0].start()
    for t in range(3):
        ih[t].start()

    def step(t):
        ih[t].wait()
        if t % 2 == 0:
            u = t // 2
            xh[u].wait()
            if u + 1 < _U0:
                if 2 * (u + 1) < _NT1:
                    xh[u + 1].start()
                else:
                    @pl.when(heavy)
                    def _():
                        xh[u + 1].start()
        if t >= 2:
            oh[t - 2].wait()
        gather(t)
        if t + 3 < _NT0:
            if t + 3 < _NT1:
                ih[t + 3].start()
            else:
                @pl.when(heavy)
                def _():
                    ih[t + 3].start()
        oh[t].start()

    for t in range(_NT1):
        step(t)

    @pl.when(light)
    def _():
        oh[_NT1 - 2].wait()
        oh[_NT1 - 1].wait()

    @pl.when(heavy)
    def _():
        for t in range(_NT1, _NT0):
            step(t)
        oh[_NT0 - 2].wait()
        oh[_NT0 - 1].wait()


@functools.partial(jax.jit, static_argnames=())
def _sampler(table, idx):
    mesh = plsc.VectorSubcoreMesh(core_axis_name="c", subcore_axis_name="s")
    call = pl.kernel(
        _sampler_body,
        out_type=jax.ShapeDtypeStruct((_B * _C, _NPOINTS), jnp.float32),
        mesh=mesh,
        scratch_types=[
            pltpu.VMEM((2 * _NPOINTS,), jnp.int32),
            pltpu.VMEM((3, _SROWS, _N), jnp.float32),
            pltpu.VMEM((2, _SROWS, _NPOINTS), jnp.float32),
            pltpu.SemaphoreType.DMA,
            pltpu.SemaphoreType.DMA,
            pltpu.SemaphoreType.DMA,
            pltpu.SemaphoreType.DMA,
            pltpu.SemaphoreType.DMA,
            pltpu.SemaphoreType.DMA,
            pltpu.SemaphoreType.DMA,
        ],
        compiler_params=pltpu.CompilerParams(needs_layout_passes=False),
    )
    return call(table, idx)


def kernel(inputs):
    table = jnp.transpose(inputs, (0, 2, 1)).reshape(_B * _C, _N)
    idx = jnp.asarray(_index_consts()).reshape(_B * _NPOINTS)
    out = _sampler(table, idx)
    return jnp.transpose(out.reshape(_B, _C, _NPOINTS), (0, 2, 1))


# speedup vs baseline: 1.1626x; 1.1475x over previous
import functools

import jax
import jax.numpy as jnp
import numpy as np
from jax import lax
from jax.experimental import pallas as pl
from jax.experimental.pallas import tpu as pltpu
from jax.experimental.pallas import tpu_sc as plsc

_B, _N, _C = 32, 8192, 64
_NPOINTS = 2048
_SROWS = 4
_NSTRIPE = _C // _SROWS

_IDX_CONST = None


def _index_consts() -> np.ndarray:
    global _IDX_CONST
    if _IDX_CONST is None:
        with jax.ensure_compile_time_eval():
            idx = jax.random.randint(
                jax.random.key(42), (_B, _NPOINTS), 0, _N, dtype=jnp.int32)
            _IDX_CONST = np.asarray(idx)
    return _IDX_CONST


def _sampler_body(table_hbm, idx_hbm, out_hbm,
                  idx_v, inbuf, outbuf, isem0, isem1, isem2, osem0, osem1):
    isems, osems = (isem0, isem1, isem2), (osem0, osem1)
    b = lax.axis_index("s") * 2 + lax.axis_index("c")
    row0 = b * _C
    rsplats = [jnp.full((16,), r, jnp.int32) for r in range(_SROWS)]

    def start_in(s):
        ph = s % 3
        return pltpu.async_copy(
            table_hbm.at[pl.ds(row0 + s * _SROWS, _SROWS)],
            inbuf.at[ph], isems[ph])

    def gather(s):
        ph = s % 3
        src = inbuf.at[ph]
        dst = outbuf.at[s % 2]

        def body(i, carry):
            base = i * 128
            idxvs = [idx_v[pl.ds(base + u * 16, 16)] for u in range(8)]
            vals = [plsc.load_gather(src, [rsplats[r], idxvs[u]])
                    for u in range(8) for r in range(_SROWS)]
            for u in range(8):
                for r in range(_SROWS):
                    dst[r, pl.ds(base + u * 16, 16)] = vals[u * _SROWS + r]
            return carry

        lax.fori_loop(0, _NPOINTS // 128, body, 0)

    ih = {}
    for t in range(3):
        ih[t] = start_in(t)
    pltpu.sync_copy(idx_hbm.at[b], idx_v)
    oh = {}
    for s in range(_NSTRIPE):
        ih[s].wait()
        if s >= 2:
            oh[s - 2].wait()
        gather(s)
        if s + 3 < _NSTRIPE:
            ih[s + 3] = start_in(s + 3)
        oh[s] = pltpu.async_copy(
            outbuf.at[s % 2],
            out_hbm.at[pl.ds(row0 + s * _SROWS, _SROWS)], osems[s % 2])
    oh[_NSTRIPE - 2].wait()
    oh[_NSTRIPE - 1].wait()


@functools.partial(jax.jit, static_argnames=())
def _sampler(table, idx):
    mesh = plsc.VectorSubcoreMesh(core_axis_name="c", subcore_axis_name="s")
    call = pl.kernel(
        _sampler_body,
        out_type=jax.ShapeDtypeStruct((_B * _C, _NPOINTS), jnp.float32),
        mesh=mesh,
        scratch_types=[
            pltpu.VMEM((_NPOINTS,), jnp.int32),
            pltpu.VMEM((3, _SROWS, _N), jnp.float32),
            pltpu.VMEM((2, _SROWS, _NPOINTS), jnp.float32),
            pltpu.SemaphoreType.DMA,
            pltpu.SemaphoreType.DMA,
            pltpu.SemaphoreType.DMA,
            pltpu.SemaphoreType.DMA,
            pltpu.SemaphoreType.DMA,
        ],
        compiler_params=pltpu.CompilerParams(needs_layout_passes=False),
    )
    return call(table, idx)


def kernel(inputs):
    table = jnp.transpose(inputs, (0, 2, 1)).reshape(_B * _C, _N)
    idx = jnp.asarray(_index_consts())
    out = _sampler(table, idx)
    return jnp.transpose(out.reshape(_B, _C, _NPOINTS), (0, 2, 1))
